# baseline (speedup 1.0000x reference)
import functools
import jax
import jax.numpy as jnp
from jax import lax
from jax.experimental import pallas as pl
from jax.experimental.pallas import tpu as pltpu
from jax.experimental.pallas import tpu_sc as plsc

B = 16384
NT = 16
CHUNK = B // NT
NV = CHUNK // 16
RADIX = 32
NPASS = 7


def _iota16():
    return lax.broadcasted_iota(jnp.int32, (16,), 0)


def _splat_sum(v):
    return jnp.full((16,), jnp.sum(v), v.dtype)


def _sc_body(d_hbm, r_hbm, e_hbm, out_hbm,
             keyA, seA, keyB, seB, totals, parts,
             dbuf, rbuf, ebuf, keybuf, sebuf, kstage, sestage, posbuf,
             hist, offs, totbuf, alltot, sbuf):
    t = lax.axis_index("s")
    base = t * CHUNK
    iota = _iota16()
    ones = jnp.full((16,), 1, jnp.int32)

    pltpu.sync_copy(d_hbm.at[pl.ds(base, CHUNK)], dbuf)
    pltpu.sync_copy(r_hbm.at[pl.ds(base, CHUNK)], rbuf)
    pltpu.sync_copy(e_hbm.at[pl.ds(base, CHUNK)], ebuf)

    def p0_body(v, acc):
        acc_er, acc_ne, acc_ss = acc
        sl = pl.ds(v * 16, 16)
        dv = dbuf[sl]
        rv = rbuf[sl]
        ev = ebuf[sl]
        u = lax.bitcast_convert_type(dv, jnp.int32)
        m = lax.shift_right_logical(u, 31)
        xorval = jnp.int32(-2147483648) + m * jnp.int32(0x7FFFFFFF)
        keybuf[sl] = (u ^ xorval) ^ jnp.int32(-1)
        se = jnp.exp(rv) * (1.0 - 2.0 * ev)
        sebuf[sl] = se
        return (acc_er + ev * rv, acc_ne + ev, acc_ss + se)

    acc_er, acc_ne, acc_ss = lax.fori_loop(
        0, NV, p0_body,
        (jnp.zeros((16,), jnp.float32), jnp.zeros((16,), jnp.float32),
         jnp.zeros((16,), jnp.float32)))
    pltpu.sync_copy(keybuf, keyA.at[pl.ds(base, CHUNK)])
    pltpu.sync_copy(sebuf, seA.at[pl.ds(base, CHUNK)])
    sbuf[pl.ds(0, 16)] = _splat_sum(acc_er)
    pltpu.sync_copy(sbuf.at[pl.ds(0, 16)], parts.at[0, t])
    sbuf[pl.ds(0, 16)] = _splat_sum(acc_ne)
    pltpu.sync_copy(sbuf.at[pl.ds(0, 16)], parts.at[1, t])
    sbuf[pl.ds(0, 16)] = _splat_sum(acc_ss)
    pltpu.sync_copy(sbuf.at[pl.ds(0, 16)], parts.at[4, t])
    plsc.subcore_barrier()

    src_k, src_s, dst_k, dst_s = keyA, seA, keyB, seB
    for p in range(NPASS):
        shift = 5 * p
        pltpu.sync_copy(src_k.at[pl.ds(base, CHUNK)], keybuf)
        pltpu.sync_copy(src_s.at[pl.ds(base, CHUNK)], sebuf)

        def zero_body(g, _):
            hist[pl.ds(g * 16, 16)] = jnp.zeros((16,), jnp.int32)
            return 0

        lax.fori_loop(0, RADIX, zero_body, 0)

        def hist_body(v, _):
            k = plsc.load_gather(keybuf, [iota * NV + v])
            dig = lax.shift_right_logical(k, shift) & (RADIX - 1)
            plsc.addupdate_scatter(hist, [dig * 16 + iota], ones)
            return 0

        lax.fori_loop(0, NV, hist_body, 0)

        def scan_body(g, _):
            sl = pl.ds(g * 16, 16)
            h = hist[sl]
            c = plsc.cumsum(h)
            offs[sl] = c - h
            plsc.store_scatter(totbuf, [jnp.full((16,), g, jnp.int32)],
                               _splat_sum(h), mask=iota == 0)
            return 0

        lax.fori_loop(0, RADIX, scan_body, 0)
        pltpu.sync_copy(totbuf, totals.at[t])
        plsc.subcore_barrier()
        pltpu.sync_copy(totals, alltot)

        def glob_body(g, grun):
            col = plsc.load_gather(alltot,
                                   [iota, jnp.full((16,), g, jnp.int32)])
            ccol = plsc.cumsum(col)
            excl = ccol - col
            mine = jnp.sum(jnp.where(iota == t, excl, 0))
            sl = pl.ds(g * 16, 16)
            offs[sl] = offs[sl] + grun + mine
            return grun + _splat_sum(col)

        lax.fori_loop(0, RADIX, glob_body, jnp.zeros((16,), jnp.int32))

        def rank_body(v, _):
            k = plsc.load_gather(keybuf, [iota * NV + v])
            s = plsc.load_gather(sebuf, [iota * NV + v])
            dig = lax.shift_right_logical(k, shift) & (RADIX - 1)
            idxv = dig * 16 + iota
            pos = plsc.load_gather(offs, [idxv])
            plsc.addupdate_scatter(offs, [idxv], ones)
            sl = pl.ds(v * 16, 16)
            kstage[sl] = k
            sestage[sl] = s
            posbuf[sl] = pos
            return 0

        lax.fori_loop(0, NV, rank_body, 0)
        pltpu.sync_copy(kstage, dst_k.at[posbuf])
        pltpu.sync_copy(sestage, dst_s.at[posbuf])
        plsc.subcore_barrier()
        src_k, src_s, dst_k, dst_s = dst_k, dst_s, src_k, src_s

    pltpu.sync_copy(src_s.at[pl.ds(base, CHUNK)], sebuf)

    def cum_body(v, acc):
        run, ssum = acc
        sl = pl.ds(v * 16, 16)
        sv = sebuf[sl]
        av = jnp.abs(sv)
        c = plsc.cumsum(av)
        dbuf[sl] = run + c
        return (run + _splat_sum(av), ssum + sv)

    run, ssum = lax.fori_loop(
        0, NV, cum_body,
        (jnp.zeros((16,), jnp.float32), jnp.zeros((16,), jnp.float32)))
    sbuf[pl.ds(0, 16)] = run
    pltpu.sync_copy(sbuf.at[pl.ds(0, 16)], parts.at[2, t])
    sbuf[pl.ds(0, 16)] = _splat_sum(ssum)
    pltpu.sync_copy(sbuf.at[pl.ds(0, 16)], parts.at[5, t])
    plsc.subcore_barrier()

    tilebase = jnp.zeros((16,), jnp.float32)
    for tt in range(NT):
        pltpu.sync_copy(parts.at[2, tt], sbuf.at[pl.ds(0, 16)])
        row = sbuf[pl.ds(0, 16)]
        w = (tt < t).astype(jnp.float32)
        tilebase = tilebase + row * w

    def log_body(v, acc):
        sl = pl.ds(v * 16, 16)
        S = dbuf[sl] + tilebase
        bitsf = lax.bitcast_convert_type(S, jnp.int32).astype(jnp.float32)
        y = (bitsf * jnp.float32(8.262958405176314e-08)
             - jnp.float32(87.989971088))
        y = y + S * jnp.exp(-y) - 1.0
        y = y + S * jnp.exp(-y) - 1.0
        ev = (sebuf[sl] < 0.0).astype(jnp.float32)
        return acc + ev * y

    acc_log = lax.fori_loop(0, NV, log_body, jnp.zeros((16,), jnp.float32))
    sbuf[pl.ds(0, 16)] = _splat_sum(acc_log)
    pltpu.sync_copy(sbuf.at[pl.ds(0, 16)], parts.at[3, t])
    plsc.subcore_barrier()

    @pl.when(t == 0)
    def _():
        er_tot = jnp.zeros((16,), jnp.float32)
        ne_tot = jnp.zeros((16,), jnp.float32)
        lg_tot = jnp.zeros((16,), jnp.float32)
        ss0_tot = jnp.zeros((16,), jnp.float32)
        ss1_tot = jnp.zeros((16,), jnp.float32)
        ab_tot = jnp.zeros((16,), jnp.float32)
        for tt in range(NT):
            pltpu.sync_copy(parts.at[0, tt], sbuf.at[pl.ds(0, 16)])
            er_tot = er_tot + sbuf[pl.ds(0, 16)]
            pltpu.sync_copy(parts.at[1, tt], sbuf.at[pl.ds(0, 16)])
            ne_tot = ne_tot + sbuf[pl.ds(0, 16)]
            pltpu.sync_copy(parts.at[3, tt], sbuf.at[pl.ds(0, 16)])
            lg_tot = lg_tot + sbuf[pl.ds(0, 16)]
            pltpu.sync_copy(parts.at[4, tt], sbuf.at[pl.ds(0, 16)])
            ss0_tot = ss0_tot + sbuf[pl.ds(0, 16)]
            pltpu.sync_copy(parts.at[5, tt], sbuf.at[pl.ds(0, 16)])
            ss1_tot = ss1_tot + sbuf[pl.ds(0, 16)]
            pltpu.sync_copy(parts.at[2, tt], sbuf.at[pl.ds(0, 16)])
            ab_tot = ab_tot + sbuf[pl.ds(0, 16)]
        pl.debug_print("ne_tot", ne_tot)
        pl.debug_print("er_tot", er_tot)
        pl.debug_print("lg_tot", lg_tot)
        pl.debug_print("ss_phase0", ss0_tot)
        pl.debug_print("ss_sorted", ss1_tot)
        pl.debug_print("abs_tot", ab_tot)
        loss = -(er_tot - lg_tot) / jnp.maximum(ne_tot, 1.0) + 1.0
        sbuf[pl.ds(0, 16)] = loss
        pltpu.sync_copy(sbuf.at[pl.ds(0, 16)], out_hbm)


@functools.partial(jax.jit, static_argnames=())
def _cox_sc(d, r, e):
    mesh = plsc.VectorSubcoreMesh(core_axis_name="c", subcore_axis_name="s",
                                  num_cores=1)
    f = pl.kernel(
        _sc_body,
        out_type=jax.ShapeDtypeStruct((16,), jnp.float32),
        mesh=mesh,
        compiler_params=pltpu.CompilerParams(needs_layout_passes=False),
        scratch_types=[
            pltpu.VMEM_SHARED((B,), jnp.int32),
            pltpu.VMEM_SHARED((B,), jnp.float32),
            pltpu.VMEM_SHARED((B,), jnp.int32),
            pltpu.VMEM_SHARED((B,), jnp.float32),
            pltpu.VMEM_SHARED((NT, RADIX), jnp.int32),
            pltpu.VMEM_SHARED((6, NT, 16), jnp.float32),
            pltpu.VMEM((CHUNK,), jnp.float32),
            pltpu.VMEM((CHUNK,), jnp.float32),
            pltpu.VMEM((CHUNK,), jnp.float32),
            pltpu.VMEM((CHUNK,), jnp.int32),
            pltpu.VMEM((CHUNK,), jnp.float32),
            pltpu.VMEM((CHUNK,), jnp.int32),
            pltpu.VMEM((CHUNK,), jnp.float32),
            pltpu.VMEM((CHUNK,), jnp.int32),
            pltpu.VMEM((RADIX * 16,), jnp.int32),
            pltpu.VMEM((RADIX * 16,), jnp.int32),
            pltpu.VMEM((RADIX,), jnp.int32),
            pltpu.VMEM((NT, RADIX), jnp.int32),
            pltpu.VMEM((16,), jnp.float32),
        ],
    )
    return f(d, r, e)


def kernel(risk_scores, targets):
    r = risk_scores
    if r.ndim > 1:
        r = jnp.squeeze(r, axis=1)
    d = targets[:, 0]
    e = targets[:, 1]
    out = _cox_sc(d, r, e)
    return out[0]


# speedup vs baseline: 5.2372x; 5.2372x over previous
import jax
import jax.numpy as jnp
from jax.experimental import pallas as pl
from jax.experimental.pallas import tpu as pltpu


def _body(d_ref, r_ref, e_ref, out_ref):
    D = d_ref[...]
    R = r_ref[...]
    E = e_ref[...]
    sub = jax.lax.broadcasted_iota(jnp.int32, (128, 1), 0)
    lane = jax.lax.broadcasted_iota(jnp.int32, (1, 128), 1)
    I = lane * 128 + sub
    IE = I.astype(jnp.float32) * 2.0 + E
    gamma = jnp.max(R)
    n_ev = jnp.sum(E)
    er_sum_term = jnp.sum(E * (R - gamma))
    ER = jnp.exp(R - gamma)

    bits = [((I >> n) & 1) for n in range(14)]
    zero_bits = jnp.zeros((128, 128), jnp.int32)
    sub_bits = [(sub & (1 << n)) != 0 for n in range(7)]
    lane_bits = [(lane & (1 << n)) != 0 for n in range(7)]

    def xor_partner(X, s, axis, bitlist):
        up = pltpu.roll(X, s, axis)
        dn = pltpu.roll(X, 128 - s, axis)
        n = s.bit_length() - 1
        return jnp.where(bitlist[n], up, dn)

    for m in range(1, 15):
        bk = bits[m] if m < 14 else zero_bits
        for j_exp in range(m - 1, -1, -1):
            bj = bits[j_exp]
            keep = bk == bj
            if j_exp <= 6:
                s = 1 << j_exp
                Dq = xor_partner(D, s, 0, sub_bits)
                IEq = xor_partner(IE, s, 0, sub_bits)
                ERq = xor_partner(ER, s, 0, sub_bits)
            else:
                s = 1 << (j_exp - 7)
                Dq = xor_partner(D, s, 1, lane_bits)
                IEq = xor_partner(IE, s, 1, lane_bits)
                ERq = xor_partner(ER, s, 1, lane_bits)
            pre = (D > Dq) | ((D == Dq) & (IE < IEq))
            take = pre == keep
            D = jnp.where(take, D, Dq)
            IE = jnp.where(take, IE, IEq)
            ER = jnp.where(take, ER, ERq)

    Lmat = (sub >= lane).astype(jnp.float32)
    colcum = jnp.dot(Lmat, ER, preferred_element_type=jnp.float32,
                     precision=jax.lax.Precision.HIGHEST)
    tot = colcum[127:128, :]
    Umat = (sub < lane).astype(jnp.float32)
    off = jnp.dot(tot, Umat, preferred_element_type=jnp.float32,
                  precision=jax.lax.Precision.HIGHEST)
    S = colcum + off

    Es = (IE.astype(jnp.int32) & 1).astype(jnp.float32)
    log_term = jnp.sum(Es * jnp.log(S))
    loss = -(er_sum_term - log_term) / jnp.maximum(n_ev, 1.0)
    out_ref[...] = jnp.full((1, 1), loss, jnp.float32)


def _cox_sorted(d_cm, r_cm, e_cm, *, interpret=False):
    return pl.pallas_call(
        _body,
        out_shape=jax.ShapeDtypeStruct((1, 1), jnp.float32),
        interpret=interpret,
    )(d_cm, r_cm, e_cm)


def kernel(risk_scores, targets, *, interpret=False):
    r = risk_scores
    if r.ndim > 1:
        r = jnp.squeeze(r, axis=1)
    d = targets[:, 0]
    e = targets[:, 1]
    d_cm = d.reshape(128, 128).T
    r_cm = r.reshape(128, 128).T
    e_cm = e.reshape(128, 128).T
    out = _cox_sorted(d_cm, r_cm, e_cm, interpret=interpret)
    return out[0, 0]
